# baseline (speedup 1.0000x reference)
import jax
import jax.numpy as jnp
import numpy as np
from jax.experimental import pallas as pl

B = 32
H = 16
KV_LEN = 4096
KV_LORA_RANK = 512
QK_ROPE_HEAD_DIM = 64
V_HEAD_DIM = 128
D_MODEL = 4096
D_LAT = KV_LORA_RANK + QK_ROPE_HEAD_DIM
SCALE = 1.0 / np.sqrt(128.0 + 64.0)

N_COL_CHUNKS = 4
COL_CHUNK = D_MODEL // N_COL_CHUNKS


def _attn_kernel(q_ref, kv_ref, o_lat_ref):
    q = q_ref[0]
    kv = kv_ref[0]

    s = jax.lax.dot_general(
        q, kv, (((1,), (1,)), ((), ())),
        preferred_element_type=jnp.float32,
    ) * SCALE
    m = jnp.max(s, axis=-1, keepdims=True)
    p = jnp.exp(s - m)
    denom = jnp.sum(p, axis=-1, keepdims=True)

    o_lat_ref[0] = jax.lax.dot_general(
        p, kv[:, :KV_LORA_RANK], (((1,), (0,)), ((), ())),
        preferred_element_type=jnp.float32,
    ) / denom


def _proj_kernel(o_lat_ref, w_uv_ref, w_o_ref, out_ref):
    o = jax.lax.dot_general(
        o_lat_ref[...], w_uv_ref[...],
        (((2,), (1,)), ((1,), (0,))),
        preferred_element_type=jnp.float32,
    )
    o = o.transpose(1, 0, 2).reshape(B, H * V_HEAD_DIM)
    out_ref[...] = jax.lax.dot_general(
        o, w_o_ref[...], (((1,), (0,)), ((), ())),
        preferred_element_type=jnp.float32,
    )


@jax.jit
def kernel(q_nope, q_pe, kv_cache, w_uv, w_o):
    q = jnp.concatenate([q_nope, q_pe], axis=-1)

    o_lat = pl.pallas_call(
        _attn_kernel,
        grid=(B,),
        in_specs=[
            pl.BlockSpec((1, H, D_LAT), lambda b: (b, 0, 0)),
            pl.BlockSpec((1, KV_LEN, D_LAT), lambda b: (b, 0, 0)),
        ],
        out_specs=pl.BlockSpec((1, H, KV_LORA_RANK), lambda b: (b, 0, 0)),
        out_shape=jax.ShapeDtypeStruct((B, H, KV_LORA_RANK), jnp.float32),
    )(q, kv_cache)

    out = pl.pallas_call(
        _proj_kernel,
        grid=(N_COL_CHUNKS,),
        in_specs=[
            pl.BlockSpec((B, H, KV_LORA_RANK), lambda c: (0, 0, 0)),
            pl.BlockSpec((H, KV_LORA_RANK, V_HEAD_DIM), lambda c: (0, 0, 0)),
            pl.BlockSpec((H * V_HEAD_DIM, COL_CHUNK), lambda c: (0, c)),
        ],
        out_specs=pl.BlockSpec((B, COL_CHUNK), lambda c: (0, c)),
        out_shape=jax.ShapeDtypeStruct((B, D_MODEL), jnp.float32),
    )(o_lat, w_uv, w_o)
    return out
